# baseline (speedup 1.0000x reference)
import functools

import jax
import jax.numpy as jnp
from jax.experimental import pallas as pl
from jax.experimental.pallas import tpu as pltpu

D = 1024
V = 8192
NL = 4
EXP = 2
CONV = 4
MLP = 4096
B = 4
L = 512
ED = EXP * D
M = B * L

_PREC = jax.lax.Precision.DEFAULT


def _dot(a, b):
    return jnp.dot(a, b, preferred_element_type=jnp.float32, precision=_PREC)


def _rms(x):
    return x * jax.lax.rsqrt(jnp.mean(x * x, axis=-1, keepdims=True) + 1e-6)


def _silu(x):
    return x * (0.5 * jnp.tanh(0.5 * x) + 0.5)


_BM = 256


def _matmul_onehot_body(ids_ref, a_ref, b_ref, o_ref):
    a = a_ref[:]
    ids = ids_ref[:]
    col = jax.lax.broadcasted_iota(jnp.int32, (_BM, V), 1)
    a = a + jnp.where(ids == col, 1.0, 0.0).astype(a.dtype)
    o_ref[:] = _dot(a, b_ref[:])


def _matmul_onehot(ids2d, a, emb):
    return pl.pallas_call(
        _matmul_onehot_body,
        grid=(M // _BM,),
        in_specs=[
            pl.BlockSpec((_BM, 1), lambda m: (m, 0)),
            pl.BlockSpec((_BM, V), lambda m: (m, 0)),
            pl.BlockSpec((V, D), lambda m: (0, 0)),
        ],
        out_specs=pl.BlockSpec((_BM, D), lambda m: (m, 0)),
        out_shape=jax.ShapeDtypeStruct((M, D), jnp.float32),
    )(ids2d, a, emb)


def _first_body(seg_ref, o_ref):
    seg = seg_ref[:]
    pos = jax.lax.broadcasted_iota(jnp.int32, (B, L), 1)
    masked = jnp.where(seg != 0, pos, L)
    o_ref[:] = jnp.minimum(jnp.min(masked, axis=1, keepdims=True), L - 1)


def _first_starts(seg2d):
    return pl.pallas_call(
        _first_body,
        out_shape=jax.ShapeDtypeStruct((B, 1), jnp.int32),
    )(seg2d)


def _shift_body(first_ref, x_ref, o_ref, pad_ref):
    b = pl.program_id(0)
    start = first_ref[b, 0]
    x = x_ref[0]
    pad_ref[0:L, :] = x
    pad_ref[L:2 * L, :] = jnp.broadcast_to(x[L - 1:L, :], (L, D))
    q8 = (start // 8) * 8
    r = start - q8
    w = pad_ref[pl.ds(q8, L + 8), :]
    o_ref[0] = pltpu.roll(w, (L + 8) - r, axis=0)[0:L, :]


def _shift(first, comb3d):
    return pl.pallas_call(
        _shift_body,
        grid=(B,),
        in_specs=[
            pl.BlockSpec(memory_space=pltpu.SMEM),
            pl.BlockSpec((1, L, D), lambda b: (b, 0, 0)),
        ],
        out_specs=pl.BlockSpec((1, L, D), lambda b: (b, 0, 0)),
        out_shape=jax.ShapeDtypeStruct((B, L, D), jnp.float32),
        scratch_shapes=[pltpu.VMEM((2 * L, D), jnp.float32)],
    )(first, comb3d)


_MC = 1024


def _mixer_compute(x, win_ref, convw_ref, wout_ref):
    h = _rms(x)
    out = x
    for c in range(ED // _MC):
        lo, hi = c * _MC, (c + 1) * _MC
        xc = _dot(h, win_ref[:, lo:hi])
        z = _dot(h, win_ref[:, ED + lo:ED + hi])
        padc = jnp.concatenate(
            [jnp.zeros((CONV - 1, _MC), jnp.float32), xc], axis=0)
        y = padc[0:L, :] * convw_ref[lo:hi, 0][None, :]
        for k in range(1, CONV):
            y = y + padc[k:k + L, :] * convw_ref[lo:hi, k][None, :]
        y = _silu(y) * _silu(z)
        out = out + _dot(y, wout_ref[lo:hi, :])
    return out


def _mixer_body(x_ref, win_ref, convw_ref, wout_ref, o_ref):
    o_ref[0] = _mixer_compute(x_ref[0], win_ref, convw_ref, wout_ref)


def _mixer0_body(first_ref, x_ref, win_ref, convw_ref, wout_ref, o_ref,
                 pad_ref):
    b = pl.program_id(0)
    start = first_ref[b, 0]
    xin = x_ref[0]
    pad_ref[0:L, :] = xin
    pad_ref[L:2 * L, :] = jnp.broadcast_to(xin[L - 1:L, :], (L, D))
    q8 = (start // 8) * 8
    r = start - q8
    w = pad_ref[pl.ds(q8, L + 8), :]
    x = pltpu.roll(w, (L + 8) - r, axis=0)[0:L, :]
    o_ref[0] = _mixer_compute(x, win_ref, convw_ref, wout_ref)


def _mixer(x3d, win_l, convw_l, wout_l, first=None):
    body = _mixer_body if first is None else _mixer0_body
    specs = [
        pl.BlockSpec((1, L, D), lambda b: (b, 0, 0)),
        pl.BlockSpec((D, 2 * ED), lambda b: (0, 0)),
        pl.BlockSpec((ED, CONV), lambda b: (0, 0)),
        pl.BlockSpec((ED, D), lambda b: (0, 0)),
    ]
    args = (x3d, win_l, convw_l, wout_l)
    scratch = []
    if first is not None:
        specs = [pl.BlockSpec(memory_space=pltpu.SMEM)] + specs
        args = (first,) + args
        scratch = [pltpu.VMEM((2 * L, D), jnp.float32)]
    return pl.pallas_call(
        body,
        grid=(B,),
        in_specs=specs,
        out_specs=pl.BlockSpec((1, L, D), lambda b: (b, 0, 0)),
        out_shape=jax.ShapeDtypeStruct((B, L, D), jnp.float32),
        scratch_shapes=scratch,
    )(*args)


_BR = 512


def _mlp_body(x_ref, w1_ref, w2_ref, o_ref):
    x = x_ref[:]
    h = _rms(x)
    a = _silu(_dot(h, w1_ref[:]))
    o_ref[:] = x + _dot(a, w2_ref[:])


def _mlp(x2d, w1_l, w2_l):
    return pl.pallas_call(
        _mlp_body,
        grid=(M // _BR,),
        in_specs=[
            pl.BlockSpec((_BR, D), lambda m: (m, 0)),
            pl.BlockSpec((D, MLP), lambda m: (0, 0)),
            pl.BlockSpec((MLP, D), lambda m: (0, 0)),
        ],
        out_specs=pl.BlockSpec((_BR, D), lambda m: (m, 0)),
        out_shape=jax.ShapeDtypeStruct((M, D), jnp.float32),
    )(x2d, w1_l, w2_l)


def _heads_body(x_ref, wmu_ref, bmu_ref, wlv_ref, blv_ref, mu_ref, lv_ref):
    x = x_ref[:]
    mu_ref[:] = _dot(x, wmu_ref[:]) + bmu_ref[:]
    lv_ref[:] = _dot(x, wlv_ref[:]) + blv_ref[:]


def _heads(x2d, wmu, bmu2d, wlv, blv2d):
    return pl.pallas_call(
        _heads_body,
        grid=(M // _BR,),
        in_specs=[
            pl.BlockSpec((_BR, D), lambda m: (m, 0)),
            pl.BlockSpec((D, D), lambda m: (0, 0)),
            pl.BlockSpec((1, D), lambda m: (0, 0)),
            pl.BlockSpec((D, D), lambda m: (0, 0)),
            pl.BlockSpec((1, D), lambda m: (0, 0)),
        ],
        out_specs=[
            pl.BlockSpec((_BR, D), lambda m: (m, 0)),
            pl.BlockSpec((_BR, D), lambda m: (m, 0)),
        ],
        out_shape=[
            jax.ShapeDtypeStruct((M, D), jnp.float32),
            jax.ShapeDtypeStruct((M, D), jnp.float32),
        ],
    )(x2d, wmu, bmu2d, wlv, blv2d)


def kernel(decoder_output, input_ids, segmentation_indices, embedding, Win,
           conv_w, Wout, W1, W2, Wmu, bmu, Wlv, blv):
    ids2d = input_ids.reshape(M, 1).astype(jnp.int32)
    seg2d = segmentation_indices.reshape(B, L).astype(jnp.int32)
    a2d = decoder_output.reshape(M, V)

    comb = _matmul_onehot(ids2d, a2d, embedding)
    first = _first_starts(seg2d)
    x = comb.reshape(B, L, D)
    for l in range(NL):
        x = _mixer(x, Win[l], conv_w[l], Wout[l],
                   first=first if l == 0 else None)
        x = _mlp(x.reshape(M, D), W1[l], W2[l]).reshape(M, D).reshape(B, L, D)
    mu, lv = _heads(x.reshape(M, D), Wmu, bmu.reshape(1, D), Wlv,
                    blv.reshape(1, D))
    return (mu.reshape(B, L, D), lv.reshape(B, L, D))


# speedup vs baseline: 1.0212x; 1.0212x over previous
import functools

import jax
import jax.numpy as jnp
from jax.experimental import pallas as pl
from jax.experimental.pallas import tpu as pltpu

D = 1024
V = 8192
NL = 4
EXP = 2
CONV = 4
MLP = 4096
B = 4
L = 512
ED = EXP * D
M = B * L

_PREC = jax.lax.Precision.DEFAULT


def _dot(a, b):
    return jnp.dot(a, b, preferred_element_type=jnp.float32, precision=_PREC)


def _rms(x):
    return x * jax.lax.rsqrt(jnp.mean(x * x, axis=-1, keepdims=True) + 1e-6)


def _silu(x):
    return x * (0.5 * jnp.tanh(0.5 * x) + 0.5)


_BM = 256


def _matmul_onehot_body(ids_ref, a_ref, b_ref, o_ref):
    a = a_ref[:]
    ids = ids_ref[:]
    col = jax.lax.broadcasted_iota(jnp.int32, (_BM, V), 1)
    a = a + jnp.where(ids == col, 1.0, 0.0).astype(a.dtype)
    o_ref[:] = _dot(a, b_ref[:])


def _matmul_onehot(ids2d, a, emb):
    return pl.pallas_call(
        _matmul_onehot_body,
        grid=(M // _BM,),
        in_specs=[
            pl.BlockSpec((_BM, 1), lambda m: (m, 0)),
            pl.BlockSpec((_BM, V), lambda m: (m, 0)),
            pl.BlockSpec((V, D), lambda m: (0, 0)),
        ],
        out_specs=pl.BlockSpec((_BM, D), lambda m: (m, 0)),
        out_shape=jax.ShapeDtypeStruct((M, D), jnp.float32),
    )(ids2d, a, emb)


def _first_body(seg_ref, o_ref):
    seg = seg_ref[:]
    pos = jax.lax.broadcasted_iota(jnp.int32, (B, L), 1)
    masked = jnp.where(seg != 0, pos, L)
    o_ref[:] = jnp.minimum(jnp.min(masked, axis=1, keepdims=True), L - 1)


def _first_starts(seg2d):
    return pl.pallas_call(
        _first_body,
        out_shape=jax.ShapeDtypeStruct((B, 1), jnp.int32),
    )(seg2d)


def _shift_body(first_ref, x_ref, o_ref, pad_ref):
    b = pl.program_id(0)
    start = first_ref[b, 0]
    x = x_ref[0]
    pad_ref[0:L, :] = x
    pad_ref[L:2 * L, :] = jnp.broadcast_to(x[L - 1:L, :], (L, D))
    q8 = (start // 8) * 8
    r = start - q8
    w = pad_ref[pl.ds(q8, L + 8), :]
    o_ref[0] = pltpu.roll(w, (L + 8) - r, axis=0)[0:L, :]


def _shift(first, comb3d):
    return pl.pallas_call(
        _shift_body,
        grid=(B,),
        in_specs=[
            pl.BlockSpec(memory_space=pltpu.SMEM),
            pl.BlockSpec((1, L, D), lambda b: (b, 0, 0)),
        ],
        out_specs=pl.BlockSpec((1, L, D), lambda b: (b, 0, 0)),
        out_shape=jax.ShapeDtypeStruct((B, L, D), jnp.float32),
        scratch_shapes=[pltpu.VMEM((2 * L, D), jnp.float32)],
    )(first, comb3d)


_MC = 1024


def _mixer_compute(x, win_ref, convw_ref, wout_ref):
    h = _rms(x)
    proj = _dot(h, win_ref[:])
    xc = proj[:, :ED]
    z = proj[:, ED:]
    padc = jnp.concatenate(
        [jnp.zeros((CONV - 1, ED), jnp.float32), xc], axis=0)
    y = padc[0:L, :] * convw_ref[:, 0][None, :]
    for k in range(1, CONV):
        y = y + padc[k:k + L, :] * convw_ref[:, k][None, :]
    y = _silu(y) * _silu(z)
    return x + _dot(y, wout_ref[:])


def _mixer_body(x_ref, win_ref, convw_ref, wout_ref, o_ref):
    o_ref[0] = _mixer_compute(x_ref[0], win_ref, convw_ref, wout_ref)


def _mixer0_body(first_ref, x_ref, win_ref, convw_ref, wout_ref, o_ref,
                 pad_ref):
    b = pl.program_id(0)
    start = first_ref[b, 0]
    xin = x_ref[0]
    pad_ref[0:L, :] = xin
    pad_ref[L:2 * L, :] = jnp.broadcast_to(xin[L - 1:L, :], (L, D))
    q8 = (start // 8) * 8
    r = start - q8
    w = pad_ref[pl.ds(q8, L + 8), :]
    x = pltpu.roll(w, (L + 8) - r, axis=0)[0:L, :]
    o_ref[0] = _mixer_compute(x, win_ref, convw_ref, wout_ref)


def _mixer(x3d, win_l, convw_l, wout_l, first=None):
    body = _mixer_body if first is None else _mixer0_body
    specs = [
        pl.BlockSpec((1, L, D), lambda b: (b, 0, 0)),
        pl.BlockSpec((D, 2 * ED), lambda b: (0, 0)),
        pl.BlockSpec((ED, CONV), lambda b: (0, 0)),
        pl.BlockSpec((ED, D), lambda b: (0, 0)),
    ]
    args = (x3d, win_l, convw_l, wout_l)
    scratch = []
    if first is not None:
        specs = [pl.BlockSpec(memory_space=pltpu.SMEM)] + specs
        args = (first,) + args
        scratch = [pltpu.VMEM((2 * L, D), jnp.float32)]
    return pl.pallas_call(
        body,
        grid=(B,),
        in_specs=specs,
        out_specs=pl.BlockSpec((1, L, D), lambda b: (b, 0, 0)),
        out_shape=jax.ShapeDtypeStruct((B, L, D), jnp.float32),
        scratch_shapes=scratch,
    )(*args)


_BR = 512


def _mlp_body(x_ref, w1_ref, w2_ref, o_ref):
    x = x_ref[:]
    h = _rms(x)
    a = _silu(_dot(h, w1_ref[:]))
    o_ref[:] = x + _dot(a, w2_ref[:])


def _mlp(x2d, w1_l, w2_l):
    return pl.pallas_call(
        _mlp_body,
        grid=(M // _BR,),
        in_specs=[
            pl.BlockSpec((_BR, D), lambda m: (m, 0)),
            pl.BlockSpec((D, MLP), lambda m: (0, 0)),
            pl.BlockSpec((MLP, D), lambda m: (0, 0)),
        ],
        out_specs=pl.BlockSpec((_BR, D), lambda m: (m, 0)),
        out_shape=jax.ShapeDtypeStruct((M, D), jnp.float32),
    )(x2d, w1_l, w2_l)


def _heads_body(x_ref, wmu_ref, bmu_ref, wlv_ref, blv_ref, mu_ref, lv_ref):
    x = x_ref[:]
    mu_ref[:] = _dot(x, wmu_ref[:]) + bmu_ref[:]
    lv_ref[:] = _dot(x, wlv_ref[:]) + blv_ref[:]


def _heads(x2d, wmu, bmu2d, wlv, blv2d):
    return pl.pallas_call(
        _heads_body,
        grid=(M // _BR,),
        in_specs=[
            pl.BlockSpec((_BR, D), lambda m: (m, 0)),
            pl.BlockSpec((D, D), lambda m: (0, 0)),
            pl.BlockSpec((1, D), lambda m: (0, 0)),
            pl.BlockSpec((D, D), lambda m: (0, 0)),
            pl.BlockSpec((1, D), lambda m: (0, 0)),
        ],
        out_specs=[
            pl.BlockSpec((_BR, D), lambda m: (m, 0)),
            pl.BlockSpec((_BR, D), lambda m: (m, 0)),
        ],
        out_shape=[
            jax.ShapeDtypeStruct((M, D), jnp.float32),
            jax.ShapeDtypeStruct((M, D), jnp.float32),
        ],
    )(x2d, wmu, bmu2d, wlv, blv2d)


def kernel(decoder_output, input_ids, segmentation_indices, embedding, Win,
           conv_w, Wout, W1, W2, Wmu, bmu, Wlv, blv):
    ids2d = input_ids.reshape(M, 1).astype(jnp.int32)
    seg2d = segmentation_indices.reshape(B, L).astype(jnp.int32)
    a2d = decoder_output.reshape(M, V)

    comb = _matmul_onehot(ids2d, a2d, embedding)
    first = _first_starts(seg2d)
    x = comb.reshape(B, L, D)
    for l in range(NL):
        x = _mixer(x, Win[l], conv_w[l], Wout[l],
                   first=first if l == 0 else None)
        x = _mlp(x.reshape(M, D), W1[l], W2[l]).reshape(M, D).reshape(B, L, D)
    mu, lv = _heads(x.reshape(M, D), Wmu, bmu.reshape(1, D), Wlv,
                    blv.reshape(1, D))
    return (mu.reshape(B, L, D), lv.reshape(B, L, D))


# speedup vs baseline: 1.0392x; 1.0176x over previous
import functools

import jax
import jax.numpy as jnp
from jax.experimental import pallas as pl
from jax.experimental.pallas import tpu as pltpu

D = 1024
V = 8192
NL = 4
EXP = 2
CONV = 4
MLP = 4096
B = 4
L = 512
ED = EXP * D
M = B * L

_PREC = jax.lax.Precision.DEFAULT


def _dot(a, b):
    return jnp.dot(a, b, preferred_element_type=jnp.float32, precision=_PREC)


def _rms(x):
    return x * jax.lax.rsqrt(jnp.mean(x * x, axis=-1, keepdims=True) + 1e-6)


def _rms_mxu(x):
    d = x.shape[-1]
    x2 = x * x
    s = jnp.dot(x2, jnp.ones((d, 128), jnp.float32),
                preferred_element_type=jnp.float32, precision=_PREC)
    s = jax.lax.rsqrt(s * (1.0 / d) + 1e-6)
    return x * jnp.concatenate([s] * (d // 128), axis=1)


def _silu(x):
    return x * (0.5 * jnp.tanh(0.5 * x) + 0.5)


_BM = 256


def _matmul_onehot_body(ids_ref, a_ref, b_ref, o_ref):
    a = a_ref[:]
    ids = ids_ref[:]
    col = jax.lax.broadcasted_iota(jnp.int32, (_BM, V), 1)
    a = a + jnp.where(ids == col, 1.0, 0.0).astype(a.dtype)
    o_ref[:] = _dot(a, b_ref[:])


def _matmul_onehot(ids2d, a, emb):
    return pl.pallas_call(
        _matmul_onehot_body,
        grid=(M // _BM,),
        in_specs=[
            pl.BlockSpec((_BM, 1), lambda m: (m, 0)),
            pl.BlockSpec((_BM, V), lambda m: (m, 0)),
            pl.BlockSpec((V, D), lambda m: (0, 0)),
        ],
        out_specs=pl.BlockSpec((_BM, D), lambda m: (m, 0)),
        out_shape=jax.ShapeDtypeStruct((M, D), jnp.float32),
    )(ids2d, a, emb)


def _first_body(seg_ref, o_ref):
    seg = seg_ref[:]
    pos = jax.lax.broadcasted_iota(jnp.int32, (B, L), 1)
    masked = jnp.where(seg != 0, pos, L)
    o_ref[:] = jnp.minimum(jnp.min(masked, axis=1, keepdims=True), L - 1)


def _first_starts(seg2d):
    return pl.pallas_call(
        _first_body,
        out_shape=jax.ShapeDtypeStruct((B, 1), jnp.int32),
    )(seg2d)


def _shift_body(first_ref, x_ref, o_ref, pad_ref):
    b = pl.program_id(0)
    start = first_ref[b, 0]
    x = x_ref[0]
    pad_ref[0:L, :] = x
    pad_ref[L:2 * L, :] = jnp.broadcast_to(x[L - 1:L, :], (L, D))
    q8 = (start // 8) * 8
    r = start - q8
    w = pad_ref[pl.ds(q8, L + 8), :]
    o_ref[0] = pltpu.roll(w, (L + 8) - r, axis=0)[0:L, :]


def _shift(first, comb3d):
    return pl.pallas_call(
        _shift_body,
        grid=(B,),
        in_specs=[
            pl.BlockSpec(memory_space=pltpu.SMEM),
            pl.BlockSpec((1, L, D), lambda b: (b, 0, 0)),
        ],
        out_specs=pl.BlockSpec((1, L, D), lambda b: (b, 0, 0)),
        out_shape=jax.ShapeDtypeStruct((B, L, D), jnp.float32),
        scratch_shapes=[pltpu.VMEM((2 * L, D), jnp.float32)],
    )(first, comb3d)


_MC = 1024


def _mixer_compute(x, win_ref, convw_ref, wout_ref, cs_ref, rms_fn=_rms):
    h = rms_fn(x)
    proj = _dot(h, win_ref[:])
    xc = proj[:, :ED]
    z = proj[:, ED:]
    cs_ref[0:8, :] = jnp.zeros((8, ED), jnp.float32)
    cs_ref[8:8 + L, :] = xc
    y = xc * convw_ref[:, CONV - 1][None, :]
    for k in range(CONV - 1):
        y = y + cs_ref[5 + k:5 + k + L, :] * convw_ref[:, k][None, :]
    y = _silu(y) * _silu(z)
    return x + _dot(y, wout_ref[:])


def _mixer_body(x_ref, win_ref, convw_ref, wout_ref, o_ref, cs_ref):
    o_ref[0] = _mixer_compute(x_ref[0], win_ref, convw_ref, wout_ref, cs_ref)


def _mixer0_body(first_ref, x_ref, win_ref, convw_ref, wout_ref, o_ref,
                 pad_ref, cs_ref):
    b = pl.program_id(0)
    start = first_ref[b, 0]
    xin = x_ref[0]
    pad_ref[0:L, :] = xin
    pad_ref[L:2 * L, :] = jnp.broadcast_to(xin[L - 1:L, :], (L, D))
    q8 = (start // 8) * 8
    r = start - q8
    w = pad_ref[pl.ds(q8, L + 8), :]
    x = pltpu.roll(w, (L + 8) - r, axis=0)[0:L, :]
    o_ref[0] = _mixer_compute(x, win_ref, convw_ref, wout_ref, cs_ref,
                              rms_fn=_rms_mxu)


def _mixer(x3d, win_l, convw_l, wout_l, first=None):
    body = _mixer_body if first is None else _mixer0_body
    specs = [
        pl.BlockSpec((1, L, D), lambda b: (b, 0, 0)),
        pl.BlockSpec((D, 2 * ED), lambda b: (0, 0)),
        pl.BlockSpec((ED, CONV), lambda b: (0, 0)),
        pl.BlockSpec((ED, D), lambda b: (0, 0)),
    ]
    args = (x3d, win_l, convw_l, wout_l)
    scratch = []
    if first is not None:
        specs = [pl.BlockSpec(memory_space=pltpu.SMEM)] + specs
        args = (first,) + args
        scratch = [pltpu.VMEM((2 * L, D), jnp.float32)]
    scratch = scratch + [pltpu.VMEM((8 + L, ED), jnp.float32)]
    return pl.pallas_call(
        body,
        grid=(B,),
        in_specs=specs,
        out_specs=pl.BlockSpec((1, L, D), lambda b: (b, 0, 0)),
        out_shape=jax.ShapeDtypeStruct((B, L, D), jnp.float32),
        scratch_shapes=scratch,
    )(*args)


_BR = 512


def _mlp_body(x_ref, w1_ref, w2_ref, o_ref):
    x = x_ref[:]
    h = _rms(x)
    a = _silu(_dot(h, w1_ref[:]))
    o_ref[:] = x + _dot(a, w2_ref[:])


def _mlp(x2d, w1_l, w2_l):
    return pl.pallas_call(
        _mlp_body,
        grid=(M // _BR,),
        in_specs=[
            pl.BlockSpec((_BR, D), lambda m: (m, 0)),
            pl.BlockSpec((D, MLP), lambda m: (0, 0)),
            pl.BlockSpec((MLP, D), lambda m: (0, 0)),
        ],
        out_specs=pl.BlockSpec((_BR, D), lambda m: (m, 0)),
        out_shape=jax.ShapeDtypeStruct((M, D), jnp.float32),
    )(x2d, w1_l, w2_l)


def _heads_body(x_ref, wmu_ref, bmu_ref, wlv_ref, blv_ref, mu_ref, lv_ref):
    x = x_ref[:]
    mu_ref[:] = _dot(x, wmu_ref[:]) + bmu_ref[:]
    lv_ref[:] = _dot(x, wlv_ref[:]) + blv_ref[:]


def _heads(x2d, wmu, bmu2d, wlv, blv2d):
    return pl.pallas_call(
        _heads_body,
        grid=(M // _BR,),
        in_specs=[
            pl.BlockSpec((_BR, D), lambda m: (m, 0)),
            pl.BlockSpec((D, D), lambda m: (0, 0)),
            pl.BlockSpec((1, D), lambda m: (0, 0)),
            pl.BlockSpec((D, D), lambda m: (0, 0)),
            pl.BlockSpec((1, D), lambda m: (0, 0)),
        ],
        out_specs=[
            pl.BlockSpec((_BR, D), lambda m: (m, 0)),
            pl.BlockSpec((_BR, D), lambda m: (m, 0)),
        ],
        out_shape=[
            jax.ShapeDtypeStruct((M, D), jnp.float32),
            jax.ShapeDtypeStruct((M, D), jnp.float32),
        ],
    )(x2d, wmu, bmu2d, wlv, blv2d)


def kernel(decoder_output, input_ids, segmentation_indices, embedding, Win,
           conv_w, Wout, W1, W2, Wmu, bmu, Wlv, blv):
    ids2d = input_ids.reshape(M, 1).astype(jnp.int32)
    seg2d = segmentation_indices.reshape(B, L).astype(jnp.int32)
    a2d = decoder_output.reshape(M, V)

    comb = _matmul_onehot(ids2d, a2d, embedding)
    first = _first_starts(seg2d)
    x = comb.reshape(B, L, D)
    for l in range(NL):
        x = _mixer(x, Win[l], conv_w[l], Wout[l],
                   first=first if l == 0 else None)
        x = _mlp(x.reshape(M, D), W1[l], W2[l]).reshape(M, D).reshape(B, L, D)
    mu, lv = _heads(x.reshape(M, D), Wmu, bmu.reshape(1, D), Wlv,
                    blv.reshape(1, D))
    return (mu.reshape(B, L, D), lv.reshape(B, L, D))
